# baseline (speedup 1.0000x reference)
import functools

import jax
import jax.numpy as jnp
from jax import lax
from jax.experimental import pallas as pl
from jax.experimental.pallas import tpu as pltpu
from jax.experimental.pallas import tpu_sc as plsc

DIM = 64
LANES = 16
CHUNK = 128


def _make_kernel(B, V):
    info = plsc.get_sparse_core_info()
    NC, NS = info.num_cores, info.num_subcores
    NW = NC * NS
    b_per_w = B // NW
    n_chunks = b_per_w // CHUNK
    mesh = plsc.VectorSubcoreMesh(core_axis_name="c", subcore_axis_name="s")

    @functools.partial(
        pl.kernel,
        mesh=mesh,
        out_type=jax.ShapeDtypeStruct((B, DIM), jnp.float32),
        compiler_params=pltpu.CompilerParams(use_tc_tiling_on_sc=False),
        scratch_types=[
            pltpu.VMEM((n_chunks, CHUNK), jnp.int32),
            pltpu.VMEM((n_chunks, CHUNK), jnp.int32),
            pltpu.VMEM((n_chunks, CHUNK), jnp.int32),
            pltpu.VMEM((b_per_w, DIM), jnp.float32),
            pltpu.VMEM((b_per_w, DIM), jnp.float32),
            pltpu.VMEM((b_per_w, DIM), jnp.float32),
            pltpu.SemaphoreType.DMA,
        ],
    )
    def k(uid_hbm, iid_hbm, cid_hbm, tu_hbm, ti_hbm, tc_hbm, out_hbm,
          iu_v, ii_v, ic_v, ubuf, ibuf, cbuf, sem):
        wid = lax.axis_index("s") * NC + lax.axis_index("c")
        row0 = wid * n_chunks
        pltpu.sync_copy(uid_hbm.at[pl.ds(row0, n_chunks)], iu_v)
        pltpu.sync_copy(iid_hbm.at[pl.ds(row0, n_chunks)], ii_v)
        pltpu.sync_copy(cid_hbm.at[pl.ds(row0, n_chunks)], ic_v)
        copies = []
        for t in range(n_chunks):
            dst = pl.ds(t * CHUNK, CHUNK)
            copies.append(pltpu.async_copy(tu_hbm.at[iu_v.at[t]], ubuf.at[dst], sem))
            copies.append(pltpu.async_copy(ti_hbm.at[ii_v.at[t]], ibuf.at[dst], sem))
            copies.append(pltpu.async_copy(tc_hbm.at[ic_v.at[t]], cbuf.at[dst], sem))
        for c in copies:
            c.wait()

        def body(i, carry):
            for j in range(DIM // LANES):
                sl = pl.ds(j * LANES, LANES)
                ubuf[i, sl] = ubuf[i, sl] + ibuf[i, sl] + cbuf[i, sl]
            return carry

        lax.fori_loop(0, b_per_w, body, 0)
        pltpu.sync_copy(ubuf, out_hbm.at[pl.ds(wid * b_per_w, b_per_w)])

    return k


def kernel(user_id, item_id, context_id, table_user, table_item, table_context, batch_size):
    B = user_id.shape[0]
    V = table_user.shape[0]
    k = _make_kernel(B, V)
    out = k(
        user_id.reshape(B // CHUNK, CHUNK),
        item_id.reshape(B // CHUNK, CHUNK),
        context_id.reshape(B // CHUNK, CHUNK),
        table_user,
        table_item,
        table_context,
    )
    return out


# speedup vs baseline: 1.8866x; 1.8866x over previous
import functools

import jax
import jax.numpy as jnp
from jax import lax
from jax.experimental import pallas as pl
from jax.experimental.pallas import tpu as pltpu
from jax.experimental.pallas import tpu_sc as plsc

DIM = 64
LANES = 16
IDX_CHUNK = 8192


def _make_kernel(B, V):
    info = plsc.get_sparse_core_info()
    NW = info.num_cores * info.num_subcores
    rows_per_w = DIM // NW
    n_idx_chunks = B // IDX_CHUNK
    mesh = plsc.VectorSubcoreMesh(core_axis_name="c", subcore_axis_name="s")

    @functools.partial(
        pl.kernel,
        mesh=mesh,
        out_type=jax.ShapeDtypeStruct((DIM, B), jnp.float32),
        compiler_params=pltpu.CompilerParams(needs_layout_passes=False),
        scratch_types=[
            pltpu.VMEM((V,), jnp.float32),
            pltpu.VMEM((IDX_CHUNK,), jnp.int32),
            pltpu.VMEM((B,), jnp.float32),
            pltpu.SemaphoreType.DMA,
        ],
    )
    def k(uid_hbm, iid_hbm, cid_hbm, tu_hbm, ti_hbm, tc_hbm, out_hbm,
          rowbuf, idxbuf, acc, sem):
        wid = lax.axis_index("s") * info.num_cores + lax.axis_index("c")

        def do_row(r, carry):
            j = wid * rows_per_w + r
            for t, (tab, ids) in enumerate(
                ((tu_hbm, uid_hbm), (ti_hbm, iid_hbm), (tc_hbm, cid_hbm))):
                pltpu.async_copy(tab.at[j], rowbuf, sem).wait()
                for ch in range(n_idx_chunks):
                    pltpu.async_copy(
                        ids.at[pl.ds(ch * IDX_CHUNK, IDX_CHUNK)], idxbuf, sem
                    ).wait()

                    def body(v, c2, _t=t, _ch=ch):
                        iv = idxbuf[pl.ds(v * LANES, LANES)]
                        g = plsc.load_gather(rowbuf, [iv])
                        sl = pl.ds(_ch * IDX_CHUNK + v * LANES, LANES)
                        if _t == 0:
                            acc[sl] = g
                        else:
                            acc[sl] = acc[sl] + g
                        return c2

                    lax.fori_loop(0, IDX_CHUNK // LANES, body, 0)
            pltpu.sync_copy(acc, out_hbm.at[j])
            return carry

        lax.fori_loop(0, rows_per_w, do_row, 0)

    return k


def kernel(user_id, item_id, context_id, table_user, table_item, table_context, batch_size):
    B = user_id.shape[0]
    V = table_user.shape[0]
    k = _make_kernel(B, V)
    out_t = k(user_id, item_id, context_id,
              table_user.T, table_item.T, table_context.T)
    return out_t.T
